# baseline (speedup 1.0000x reference)
import functools

import jax
import jax.numpy as jnp
from jax import lax
from jax.experimental import pallas as pl
from jax.experimental.pallas import tpu as pltpu
from jax.experimental.pallas import tpu_sc as plsc


def _sc_gather(idx, table):
    BC = idx.shape[0]
    V, D = table.shape
    info = plsc.get_sparse_core_info()
    NW = info.num_cores * info.num_subcores
    per_w = BC // NW
    CH = 8
    n_ch = per_w // CH
    mesh = plsc.VectorSubcoreMesh(core_axis_name="c", subcore_axis_name="s")

    @functools.partial(
        pl.kernel,
        mesh=mesh,
        out_type=jax.ShapeDtypeStruct((BC, D), jnp.float32),
        scratch_types=[
            pltpu.VMEM((2, CH), jnp.int32),
            pltpu.VMEM((2, CH, D), jnp.float32),
            pltpu.SemaphoreType.DMA((2,)),
            pltpu.SemaphoreType.DMA((2,)),
        ],
    )
    def gk(idx_hbm, table_hbm, out_hbm, idx_v, rows_v, gsem, wsem):
        wid = lax.axis_index("s") * info.num_cores + lax.axis_index("c")
        base = wid * per_w

        def out_at(k):
            return out_hbm.at[pl.ds(base + k * CH, CH)]

        for k in range(n_ch):
            b = k % 2
            if k >= 2:
                pltpu.make_async_copy(rows_v.at[b], out_at(k - 2),
                                      wsem.at[b]).wait()
            pltpu.sync_copy(idx_hbm.at[pl.ds(base + k * CH, CH)], idx_v.at[b])
            pltpu.async_copy(table_hbm.at[idx_v.at[b]], rows_v.at[b],
                             gsem.at[b])
            if k >= 1:
                bb = (k - 1) % 2
                pltpu.make_async_copy(table_hbm.at[idx_v.at[bb]],
                                      rows_v.at[bb], gsem.at[bb]).wait()
                pltpu.async_copy(rows_v.at[bb], out_at(k - 1), wsem.at[bb])
        bl = (n_ch - 1) % 2
        pltpu.make_async_copy(table_hbm.at[idx_v.at[bl]], rows_v.at[bl],
                              gsem.at[bl]).wait()
        pltpu.async_copy(rows_v.at[bl], out_at(n_ch - 1), wsem.at[bl])
        pltpu.make_async_copy(rows_v.at[1 - bl], out_at(n_ch - 2),
                              wsem.at[1 - bl]).wait()
        pltpu.make_async_copy(rows_v.at[bl], out_at(n_ch - 1),
                              wsem.at[bl]).wait()

    return gk(idx, table)


_BF = jnp.bfloat16
_F8 = jnp.float8_e4m3fn
_SCALE = 256.0
_INV = 1.0 / _SCALE
_NT = (((1,), (1,)), ((), ()))


def _e_prefetch_map(C, nm):
    def emap(c, m, j):
        sid = c * nm + m
        sid = jnp.where(j >= 2, jnp.minimum(sid + 1, C * nm - 1), sid)
        return (sid // nm, sid % nm, 0)
    return emap


def _layer1a(e0, w1, nm=2, nb=512):
    B, D = e0.shape
    H = w1.shape[0]
    mb = B // nm
    n_nb = H // nb
    hk = D // 2

    def body(x_ref, wa_ref, wb_ref, o_ref, xq_ref):
        j = pl.program_id(1)

        @pl.when(j == 0)
        def _():
            xq_ref[...] = (x_ref[...] * _SCALE).astype(_F8)

        d = lax.dot_general(xq_ref[:, :hk], wa_ref[...].astype(_F8), _NT,
                            preferred_element_type=jnp.float32)
        d += lax.dot_general(xq_ref[:, hk:], wb_ref[...].astype(_F8), _NT,
                             preferred_element_type=jnp.float32)
        o_ref[...] = d.astype(_BF)

    def emap(m, j):
        mm = jnp.where(j >= 2, jnp.minimum(m + 1, nm - 1), m)
        return (mm, 0)

    return pl.pallas_call(
        body,
        grid=(nm, n_nb),
        in_specs=[
            pl.BlockSpec((mb, D), emap),
            pl.BlockSpec((nb, hk), lambda m, j: (j, 0)),
            pl.BlockSpec((nb, hk), lambda m, j: (j, 1)),
        ],
        out_specs=pl.BlockSpec((mb, nb), lambda m, j: (m, j)),
        out_shape=jax.ShapeDtypeStruct((B, H), _BF),
        scratch_shapes=[pltpu.VMEM((mb, D), _F8)],
    )(e0, w1, w1)


def _layer1b(e3, w1, b1, acc0, nm=2, nb=512):
    C, B, D = e3.shape
    H = w1.shape[0]
    mb = B // nm
    n_nb = H // nb
    hk = D // 2

    def body(x_ref, wa_ref, wb_ref, b_ref, a0_ref, o_ref, xq_ref, acc_ref):
        c = pl.program_id(0)
        m = pl.program_id(1)
        j = pl.program_id(2)

        @pl.when(j == 0)
        def _():
            xq_ref[...] = (x_ref[0] * _SCALE).astype(_F8)

        d = lax.dot_general(xq_ref[:, :hk], wa_ref[...].astype(_F8), _NT,
                            preferred_element_type=jnp.float32)
        d += lax.dot_general(xq_ref[:, hk:], wb_ref[...].astype(_F8), _NT,
                             preferred_element_type=jnp.float32)

        @pl.when(c == 0)
        def _():
            acc_ref[m, j] = (a0_ref[...].astype(jnp.float32) + d).astype(_BF)

        @pl.when(jnp.logical_and(c > 0, c < C - 1))
        def _():
            acc_ref[m, j] += d.astype(_BF)

        @pl.when(c == C - 1)
        def _():
            z = acc_ref[m, j].astype(jnp.float32) + d + b_ref[...] * _SCALE
            o_ref[...] = jnp.maximum(z, 0.0).astype(_F8)

    last = C - 1
    return pl.pallas_call(
        body,
        grid=(C, nm, n_nb),
        in_specs=[
            pl.BlockSpec((1, mb, D), _e_prefetch_map(C, nm)),
            pl.BlockSpec((nb, hk), lambda c, m, j: (j, 2 * (c + 1))),
            pl.BlockSpec((nb, hk), lambda c, m, j: (j, 2 * (c + 1) + 1)),
            pl.BlockSpec((1, nb), lambda c, m, j: (0, j)),
            pl.BlockSpec((mb, nb),
                         lambda c, m, j: (jnp.where(c == 0, m, 0),
                                          jnp.where(c == 0, j, 0))),
        ],
        out_specs=pl.BlockSpec(
            (mb, nb),
            lambda c, m, j: (jnp.where(c == last, m, 0),
                             jnp.where(c == last, j, 0))),
        out_shape=jax.ShapeDtypeStruct((B, H), _F8),
        scratch_shapes=[
            pltpu.VMEM((mb, D), _F8),
            pltpu.VMEM((nm, n_nb, mb, nb), _BF),
        ],
    )(e3, w1, w1, b1, acc0)


def _layer1(e3, w1, b1, nm=2, nb=512):
    C, B, D = e3.shape
    H = w1.shape[0]
    mb = B // nm
    n_nb = H // nb
    hk = D // 2

    def body(x_ref, wa_ref, wb_ref, b_ref, o_ref, xq_ref, acc_ref):
        c = pl.program_id(0)
        m = pl.program_id(1)
        j = pl.program_id(2)

        @pl.when(j == 0)
        def _():
            xq_ref[...] = (x_ref[0] * _SCALE).astype(_F8)

        d = lax.dot_general(xq_ref[:, :hk], wa_ref[...].astype(_F8), _NT,
                            preferred_element_type=jnp.float32)
        d += lax.dot_general(xq_ref[:, hk:], wb_ref[...].astype(_F8), _NT,
                             preferred_element_type=jnp.float32)

        @pl.when(c == 0)
        def _():
            acc_ref[m, j] = d.astype(_BF)

        @pl.when(jnp.logical_and(c > 0, c < C - 1))
        def _():
            acc_ref[m, j] += d.astype(_BF)

        @pl.when(c == C - 1)
        def _():
            z = acc_ref[m, j].astype(jnp.float32) + d + b_ref[...] * _SCALE
            o_ref[...] = jnp.maximum(z, 0.0).astype(_F8)

    last = C - 1
    return pl.pallas_call(
        body,
        grid=(C, nm, n_nb),
        in_specs=[
            pl.BlockSpec((1, mb, D), _e_prefetch_map(C, nm)),
            pl.BlockSpec((nb, hk), lambda c, m, j: (j, 2 * c)),
            pl.BlockSpec((nb, hk), lambda c, m, j: (j, 2 * c + 1)),
            pl.BlockSpec((1, nb), lambda c, m, j: (0, j)),
        ],
        out_specs=pl.BlockSpec(
            (mb, nb),
            lambda c, m, j: (jnp.where(c == last, m, 0),
                             jnp.where(c == last, j, 0))),
        out_shape=jax.ShapeDtypeStruct((B, H), _F8),
        scratch_shapes=[
            pltpu.VMEM((mb, D), _F8),
            pltpu.VMEM((nm, n_nb, mb, nb), _BF),
        ],
    )(e3, w1, w1, b1)


def _layer_stream(xq, w, b, out_kind, nb=512):
    M, K = xq.shape
    N = w.shape[0]
    hk = K // 2

    def body(x_ref, wa_ref, wb_ref, b_ref, o_ref):
        z = lax.dot_general(x_ref[:, :hk], wa_ref[...].astype(_F8), _NT,
                            preferred_element_type=jnp.float32)
        z += lax.dot_general(x_ref[:, hk:], wb_ref[...].astype(_F8), _NT,
                             preferred_element_type=jnp.float32)
        if out_kind == "f8":
            o_ref[...] = jnp.maximum(z + b_ref[...] * _SCALE, 0.0).astype(_F8)
        else:
            o_ref[...] = (z * _INV + b_ref[...]).astype(_BF)

    return pl.pallas_call(
        body,
        grid=(N // nb,),
        in_specs=[
            pl.BlockSpec((M, K), lambda j: (0, 0)),
            pl.BlockSpec((nb, hk), lambda j: (j, 0)),
            pl.BlockSpec((nb, hk), lambda j: (j, 1)),
            pl.BlockSpec((1, nb), lambda j: (0, j)),
        ],
        out_specs=pl.BlockSpec((M, nb), lambda j: (0, j)),
        out_shape=jax.ShapeDtypeStruct(
            (M, N), _F8 if out_kind == "f8" else _BF),
    )(xq, w, w, b)


def _log_softmax(z, mb=256):
    M, N = z.shape

    def body(z_ref, o_ref):
        zz = z_ref[...].astype(jnp.float32)
        m = jnp.max(zz, axis=1, keepdims=True)
        zs = zz - m
        s = jnp.sum(jnp.exp(zs), axis=1, keepdims=True)
        o_ref[...] = zs - jnp.log(s)

    return pl.pallas_call(
        body,
        grid=(M // mb,),
        in_specs=[pl.BlockSpec((mb, N), lambda i: (i, 0))],
        out_specs=pl.BlockSpec((mb, N), lambda i: (i, 0)),
        out_shape=jax.ShapeDtypeStruct((M, N), jnp.float32),
    )(z)


def kernel(x, table, W1, b1, W2, b2, W3, b3):
    B, C = x.shape
    V, D = table.shape
    idx = x.T.reshape(-1).astype(jnp.int32)
    e0 = _sc_gather(idx[:B], table)
    er = _sc_gather(idx[B:], table).reshape(C - 1, B, D)
    acc0 = _layer1a(e0, W1)
    h1 = _layer1b(er, W1, b1.reshape(1, -1), acc0)
    h2 = _layer_stream(h1, W2, b2.reshape(1, -1), out_kind="f8")
    z = _layer_stream(h2, W3, b3.reshape(1, -1), out_kind="bf16")
    return _log_softmax(z)


# speedup vs baseline: 1.1985x; 1.1985x over previous
import functools

import jax
import jax.numpy as jnp
from jax import lax
from jax.experimental import pallas as pl
from jax.experimental.pallas import tpu as pltpu
from jax.experimental.pallas import tpu_sc as plsc


def _sc_gather(idx, table):
    BC = idx.shape[0]
    V, D = table.shape
    info = plsc.get_sparse_core_info()
    NW = info.num_cores * info.num_subcores
    per_w = BC // NW
    CH = 8
    n_ch = per_w // CH
    mesh = plsc.VectorSubcoreMesh(core_axis_name="c", subcore_axis_name="s")

    @functools.partial(
        pl.kernel,
        mesh=mesh,
        out_type=jax.ShapeDtypeStruct((BC, D), jnp.float32),
        scratch_types=[
            pltpu.VMEM((2, CH), jnp.int32),
            pltpu.VMEM((2, CH, D), jnp.float32),
            pltpu.SemaphoreType.DMA((2,)),
            pltpu.SemaphoreType.DMA((2,)),
        ],
    )
    def gk(idx_hbm, table_hbm, out_hbm, idx_v, rows_v, gsem, wsem):
        wid = lax.axis_index("s") * info.num_cores + lax.axis_index("c")
        base = wid * per_w

        def out_at(k):
            return out_hbm.at[pl.ds(base + k * CH, CH)]

        for k in range(n_ch):
            b = k % 2
            if k >= 2:
                pltpu.make_async_copy(rows_v.at[b], out_at(k - 2),
                                      wsem.at[b]).wait()
            pltpu.sync_copy(idx_hbm.at[pl.ds(base + k * CH, CH)], idx_v.at[b])
            pltpu.async_copy(table_hbm.at[idx_v.at[b]], rows_v.at[b],
                             gsem.at[b])
            if k >= 1:
                bb = (k - 1) % 2
                pltpu.make_async_copy(table_hbm.at[idx_v.at[bb]],
                                      rows_v.at[bb], gsem.at[bb]).wait()
                pltpu.async_copy(rows_v.at[bb], out_at(k - 1), wsem.at[bb])
        bl = (n_ch - 1) % 2
        pltpu.make_async_copy(table_hbm.at[idx_v.at[bl]], rows_v.at[bl],
                              gsem.at[bl]).wait()
        pltpu.async_copy(rows_v.at[bl], out_at(n_ch - 1), wsem.at[bl])
        pltpu.make_async_copy(rows_v.at[1 - bl], out_at(n_ch - 2),
                              wsem.at[1 - bl]).wait()
        pltpu.make_async_copy(rows_v.at[bl], out_at(n_ch - 1),
                              wsem.at[bl]).wait()

    return gk(idx, table)


_BF = jnp.bfloat16
_F8 = jnp.float8_e4m3fn
_SCALE = 256.0
_INV = 1.0 / _SCALE
_NT = (((1,), (1,)), ((), ()))


def _e_prefetch_map(C, nm):
    del nm
    def emap(c, j):
        cc = jnp.where(j >= 2, jnp.minimum(c + 1, C - 1), c)
        return (cc, 0, 0)
    return emap


def _layer1a(e01, w1, nb=256):
    CA, B, D = e01.shape
    H = w1.shape[0]
    n_nb = H // nb
    hk = D // 2

    def body(x_ref, wa_ref, wb_ref, o_ref, xq_ref, acc_ref):
        c = pl.program_id(0)
        j = pl.program_id(1)

        @pl.when(j == 0)
        def _():
            xq_ref[...] = (x_ref[0] * _SCALE).astype(_F8)

        d = lax.dot_general(xq_ref[:, :hk], wa_ref[...].astype(_F8), _NT,
                            preferred_element_type=jnp.float32)
        d += lax.dot_general(xq_ref[:, hk:], wb_ref[...].astype(_F8), _NT,
                             preferred_element_type=jnp.float32)

        @pl.when(c == 0)
        def _():
            acc_ref[j] = d.astype(_BF)

        @pl.when(c == CA - 1)
        def _():
            o_ref[...] = (acc_ref[j].astype(jnp.float32) + d).astype(_BF)

    last = CA - 1
    return pl.pallas_call(
        body,
        grid=(CA, n_nb),
        in_specs=[
            pl.BlockSpec((1, B, D), _e_prefetch_map(CA, 1)),
            pl.BlockSpec((nb, hk), lambda c, j: (j, 2 * c)),
            pl.BlockSpec((nb, hk), lambda c, j: (j, 2 * c + 1)),
        ],
        out_specs=pl.BlockSpec(
            (B, nb), lambda c, j: (0, jnp.where(c == last, j, 0))),
        out_shape=jax.ShapeDtypeStruct((B, H), _BF),
        scratch_shapes=[
            pltpu.VMEM((B, D), _F8),
            pltpu.VMEM((n_nb, B, nb), _BF),
        ],
    )(e01, w1, w1)


def _layer1b(e3, w1, b1, acc0, c_off=2, nb=256):
    C, B, D = e3.shape
    H = w1.shape[0]
    n_nb = H // nb
    hk = D // 2

    def body(x_ref, wa_ref, wb_ref, b_ref, a0_ref, o_ref, xq_ref, acc_ref):
        c = pl.program_id(0)
        j = pl.program_id(1)

        @pl.when(j == 0)
        def _():
            xq_ref[...] = (x_ref[0] * _SCALE).astype(_F8)

        d = lax.dot_general(xq_ref[:, :hk], wa_ref[...].astype(_F8), _NT,
                            preferred_element_type=jnp.float32)
        d += lax.dot_general(xq_ref[:, hk:], wb_ref[...].astype(_F8), _NT,
                             preferred_element_type=jnp.float32)

        @pl.when(c == 0)
        def _():
            acc_ref[j] = (a0_ref[...].astype(jnp.float32) + d).astype(_BF)

        @pl.when(jnp.logical_and(c > 0, c < C - 1))
        def _():
            acc_ref[j] += d.astype(_BF)

        @pl.when(c == C - 1)
        def _():
            z = acc_ref[j].astype(jnp.float32) + d + b_ref[...] * _SCALE
            o_ref[...] = jnp.maximum(z, 0.0).astype(_F8)

    last = C - 1
    return pl.pallas_call(
        body,
        grid=(C, n_nb),
        in_specs=[
            pl.BlockSpec((1, B, D), _e_prefetch_map(C, 1)),
            pl.BlockSpec((nb, hk), lambda c, j: (j, 2 * (c + c_off))),
            pl.BlockSpec((nb, hk), lambda c, j: (j, 2 * (c + c_off) + 1)),
            pl.BlockSpec((1, nb), lambda c, j: (0, j)),
            pl.BlockSpec((B, nb),
                         lambda c, j: (0, jnp.where(c == 0, j, 0))),
        ],
        out_specs=pl.BlockSpec(
            (B, nb), lambda c, j: (0, jnp.where(c == last, j, 0))),
        out_shape=jax.ShapeDtypeStruct((B, H), _F8),
        scratch_shapes=[
            pltpu.VMEM((B, D), _F8),
            pltpu.VMEM((n_nb, B, nb), _BF),
        ],
    )(e3, w1, w1, b1, acc0)


def _layer_stream(xq, w, b, out_kind, nb=512):
    M, K = xq.shape
    N = w.shape[0]
    hk = K // 2

    def body(x_ref, wa_ref, wb_ref, b_ref, o_ref):
        z = lax.dot_general(x_ref[:, :hk], wa_ref[...].astype(_F8), _NT,
                            preferred_element_type=jnp.float32)
        z += lax.dot_general(x_ref[:, hk:], wb_ref[...].astype(_F8), _NT,
                             preferred_element_type=jnp.float32)
        if out_kind == "f8":
            o_ref[...] = jnp.maximum(z + b_ref[...] * _SCALE, 0.0).astype(_F8)
        else:
            o_ref[...] = (z * _INV + b_ref[...]).astype(_BF)

    return pl.pallas_call(
        body,
        grid=(N // nb,),
        in_specs=[
            pl.BlockSpec((M, K), lambda j: (0, 0)),
            pl.BlockSpec((nb, hk), lambda j: (j, 0)),
            pl.BlockSpec((nb, hk), lambda j: (j, 1)),
            pl.BlockSpec((1, nb), lambda j: (0, j)),
        ],
        out_specs=pl.BlockSpec((M, nb), lambda j: (0, j)),
        out_shape=jax.ShapeDtypeStruct(
            (M, N), _F8 if out_kind == "f8" else _BF),
    )(xq, w, w, b)


def _log_softmax(z, mb=256):
    M, N = z.shape

    def body(z_ref, o_ref):
        zz = z_ref[...].astype(jnp.float32)
        m = jnp.max(zz, axis=1, keepdims=True)
        zs = zz - m
        s = jnp.sum(jnp.exp(zs), axis=1, keepdims=True)
        o_ref[...] = zs - jnp.log(s)

    return pl.pallas_call(
        body,
        grid=(M // mb,),
        in_specs=[pl.BlockSpec((mb, N), lambda i: (i, 0))],
        out_specs=pl.BlockSpec((mb, N), lambda i: (i, 0)),
        out_shape=jax.ShapeDtypeStruct((M, N), jnp.float32),
    )(z)


def kernel(x, table, W1, b1, W2, b2, W3, b3):
    B, C = x.shape
    V, D = table.shape
    idx = x.T.reshape(-1).astype(jnp.int32)
    e01 = _sc_gather(idx[:2 * B], table).reshape(2, B, D)
    er = _sc_gather(idx[2 * B:], table).reshape(C - 2, B, D)
    acc0 = _layer1a(e01, W1)
    h1 = _layer1b(er, W1, b1.reshape(1, -1), acc0)
    h2 = _layer_stream(h1, W2, b2.reshape(1, -1), out_kind="f8")
    z = _layer_stream(h2, W3, b3.reshape(1, -1), out_kind="bf16")
    return _log_softmax(z)
